# baseline (speedup 1.0000x reference)
import functools

import jax
import jax.numpy as jnp
from jax import lax
from jax.experimental import pallas as pl
from jax.experimental.pallas import tpu as pltpu
from jax.experimental.pallas import tpu_sc as plsc

_N = 10000
_NP = 10240
_NH = 5120
_NHA = 5136
_E = 160000
_ER = 1250
_EC = 128
_HID = 1024
_NS = 16
_TPB = _NP // _NS
_TE = 640

_f32 = jnp.float32
_SC_PARAMS = pltpu.CompilerParams(use_tc_tiling_on_sc=False,
                                  needs_layout_passes=False)


def _sc_mesh():
  return plsc.VectorSubcoreMesh(core_axis_name="c", subcore_axis_name="s")


def _sc_gather_body(feat_hbm, colg_hbm, rowg_hbm, ewg_hbm, zrow_hbm,
                    c16_hbm, r16_hbm, degp_hbm,
                    idxc_v, idxr_v, crows_v, rrows_v, ew_v, z_v, acc_sh, sem):
  cid = lax.axis_index("c")
  sid = lax.axis_index("s")
  pltpu.sync_copy(zrow_hbm, z_v)
  pltpu.sync_copy(z_v, acc_sh.at[pl.ds(sid * _TPB, _TPB)])
  plsc.subcore_barrier()

  def body(g, carry):
    lr = sid + g * _NS

    @pl.when(lr < _ER // 2)
    def _():
      r = cid * (_ER // 2) + lr
      pltpu.sync_copy(colg_hbm.at[r], idxc_v)
      pltpu.sync_copy(rowg_hbm.at[r], idxr_v)
      pltpu.async_copy(feat_hbm.at[idxc_v], crows_v, sem).wait()
      pltpu.sync_copy(crows_v, c16_hbm.at[pl.ds(r * _EC, _EC)])
      pltpu.async_copy(feat_hbm.at[idxr_v], rrows_v, sem).wait()
      pltpu.sync_copy(rrows_v, r16_hbm.at[pl.ds(r * _EC, _EC)])
      pltpu.sync_copy(ewg_hbm.at[r], ew_v)
      pltpu.sync_copy(ew_v, acc_sh.at[idxc_v], add=True)
    return carry

  lax.fori_loop(0, (_ER // 2 + _NS - 1) // _NS, body, 0)
  plsc.subcore_barrier()
  pltpu.sync_copy(acc_sh.at[pl.ds(sid * _TPB, _TPB)],
                  degp_hbm.at[cid, pl.ds(sid * _TPB, _TPB)])


def _sc_msg_body(mlo_hbm, mhi_hbm, colg_hbm, zblk_hbm,
                 h0lo_hbm, h0hi_hbm,
                 idx_v, mrow_v, z_v, acc_sh, sem):
  cid = lax.axis_index("c")
  sid = lax.axis_index("s")
  rows_per_tile = _NHA // _NS
  pltpu.sync_copy(zblk_hbm.at[pl.ds(0, rows_per_tile)], z_v)

  for p in range(2):
    base = p * _NH
    pltpu.sync_copy(z_v, acc_sh.at[pl.ds(sid * rows_per_tile, rows_per_tile)])
    plsc.subcore_barrier()

    def body(g, carry):
      r = sid + g * _NS

      @pl.when(r < _ER)
      def _():
        pltpu.sync_copy(colg_hbm.at[r], idx_v)
        for k in range(_EC // 16):
          v = idx_v[pl.ds(k * 16, 16)] - base
          ok = jnp.logical_and(v >= 0, v < _NH)
          idx_v[pl.ds(k * 16, 16)] = jnp.where(ok, v, _NH)

        @pl.when(cid == 0)
        def _():
          pltpu.sync_copy(mlo_hbm.at[pl.ds(r * _EC, _EC)], mrow_v)

        @pl.when(cid == 1)
        def _():
          pltpu.sync_copy(mhi_hbm.at[pl.ds(r * _EC, _EC)], mrow_v)

        pltpu.sync_copy(mrow_v, acc_sh.at[idx_v], add=True)
      return carry

    lax.fori_loop(0, (_ER + _NS - 1) // _NS, body, 0)
    plsc.subcore_barrier()
    out_rows = _NH // _NS
    src = pl.ds(sid * out_rows, out_rows)
    dst = pl.ds(base + sid * out_rows, out_rows)

    @pl.when(cid == 0)
    def _():
      pltpu.sync_copy(acc_sh.at[src], h0lo_hbm.at[dst])

    @pl.when(cid == 1)
    def _():
      pltpu.sync_copy(acc_sh.at[src], h0hi_hbm.at[dst])

    plsc.subcore_barrier()


def _sc_spmm_body(dh, glo_hbm, ghi_hbm, colg_hbm, rowg_hbm, ewg_hbm,
                  zblk_hbm, tlo_hbm, thi_hbm,
                  idxc_v, idxr_v, grow_v, ew_v, z_v, acc_sh, sem):
  cid = lax.axis_index("c")
  sid = lax.axis_index("s")
  pltpu.sync_copy(zblk_hbm.at[:, pl.ds(0, dh)], z_v)
  pltpu.sync_copy(z_v, acc_sh.at[pl.ds(sid * _TPB, _TPB)])
  plsc.subcore_barrier()

  def body(g, carry):
    r = sid + g * _NS

    @pl.when(r < _ER)
    def _():
      pltpu.sync_copy(colg_hbm.at[r], idxc_v)
      pltpu.sync_copy(rowg_hbm.at[r], idxr_v)
      pltpu.sync_copy(ewg_hbm.at[r], ew_v)

      @pl.when(cid == 0)
      def _():
        pltpu.async_copy(glo_hbm.at[idxr_v], grow_v, sem).wait()

      @pl.when(cid == 1)
      def _():
        pltpu.async_copy(ghi_hbm.at[idxr_v], grow_v, sem).wait()

      def escale(e, c2):
        w = plsc.load_gather(ew_v, [jnp.zeros((16,), jnp.int32) + e])
        for j in range(dh // 16):
          grow_v[e, pl.ds(j * 16, 16)] = grow_v[e, pl.ds(j * 16, 16)] * w
        return c2

      lax.fori_loop(0, _EC, escale, 0)
      pltpu.sync_copy(grow_v, acc_sh.at[idxc_v], add=True)
    return carry

  lax.fori_loop(0, (_ER + _NS - 1) // _NS, body, 0)
  plsc.subcore_barrier()
  sl = pl.ds(sid * _TPB, _TPB)

  @pl.when(cid == 0)
  def _():
    pltpu.sync_copy(acc_sh.at[sl], tlo_hbm.at[sl])

  @pl.when(cid == 1)
  def _():
    pltpu.sync_copy(acc_sh.at[sl], thi_hbm.at[sl])


def _dinv_body(degp_ref, out_ref):
  deg = degp_ref[0] + degp_ref[1]
  out_ref[...] = jnp.where(deg > 0, lax.rsqrt(jnp.maximum(deg, 1e-30)), 0.0)


def _phi_body(c16_ref, r16_ref, ew_ref, w0c_ref, w0r_ref, w0e_ref, b0_ref,
              w1_ref, b1_ref, w2_ref, b2_ref, w3lo_ref, w3hi_ref,
              b3lo_ref, b3hi_ref, mlo_ref, mhi_ref):
  h = jnp.dot(c16_ref[...], w0c_ref[...], preferred_element_type=_f32)
  h = h + jnp.dot(r16_ref[...], w0r_ref[...], preferred_element_type=_f32)
  h = h + ew_ref[...] * w0e_ref[...]
  h = jax.nn.relu(h + b0_ref[...]).astype(jnp.bfloat16)
  h = jax.nn.relu(jnp.dot(h, w1_ref[...], preferred_element_type=_f32)
                  + b1_ref[...]).astype(jnp.bfloat16)
  h = jax.nn.relu(jnp.dot(h, w2_ref[...], preferred_element_type=_f32)
                  + b2_ref[...]).astype(jnp.bfloat16)
  mlo_ref[...] = jnp.dot(h, w3lo_ref[...], preferred_element_type=_f32) \
      + b3lo_ref[...]
  mhi_ref[...] = jnp.dot(h, w3hi_ref[...], preferred_element_type=_f32) \
      + b3hi_ref[...]


def _g1_body(h0lo_ref, h0hi_ref, dinv_ref, *refs):
  wlo, whi, outs = refs[:4], refs[4:8], refs[8:]
  lo, hi = h0lo_ref[...], h0hi_ref[...]
  d = dinv_ref[...]
  for q in range(4):
    outs[q][...] = d * (
        jnp.dot(lo, wlo[q][...], preferred_element_type=_f32)
        + jnp.dot(hi, whi[q][...], preferred_element_type=_f32))


def _g2_body(t1a_ref, t1b_ref, t1c_ref, t1d_ref, dinv_ref,
             b1a_ref, b1b_ref, b1c_ref, b1d_ref,
             w2a_lo, w2b_lo, w2c_lo, w2d_lo, w2a_hi, w2b_hi, w2c_hi, w2d_hi,
             glo_ref, ghi_ref):
  d = dinv_ref[...]
  h1 = [jax.nn.relu(d * t[...] + b[...])
        for t, b in ((t1a_ref, b1a_ref), (t1b_ref, b1b_ref),
                     (t1c_ref, b1c_ref), (t1d_ref, b1d_ref))]
  wlo = (w2a_lo, w2b_lo, w2c_lo, w2d_lo)
  whi = (w2a_hi, w2b_hi, w2c_hi, w2d_hi)
  glo = jnp.dot(h1[0], wlo[0][...], preferred_element_type=_f32)
  ghi = jnp.dot(h1[0], whi[0][...], preferred_element_type=_f32)
  for q in range(1, 4):
    glo = glo + jnp.dot(h1[q], wlo[q][...], preferred_element_type=_f32)
    ghi = ghi + jnp.dot(h1[q], whi[q][...], preferred_element_type=_f32)
  glo_ref[...] = d * glo
  ghi_ref[...] = d * ghi


def _out_body(t2lo_ref, t2hi_ref, dinv_ref, s0_ref, s1_ref, b2_ref, out_ref):
  d = dinv_ref[...]
  t2 = jnp.dot(t2lo_ref[...], s0_ref[...], preferred_element_type=_f32) \
      + jnp.dot(t2hi_ref[...], s1_ref[...], preferred_element_type=_f32)
  out_ref[...] = jax.nn.relu(d * t2 + b2_ref[...])


def _full(shape):
  return pl.BlockSpec(shape, lambda *_: tuple(0 for _ in shape))


def _pad(a, shape):
  out = jnp.zeros(shape, _f32)
  return out.at[tuple(slice(0, s) for s in a.shape)].set(a)


def kernel(x, edge_attr, edge_index, emb_table,
           phi_w0, phi_b0, phi_w1, phi_b1, phi_w2, phi_b2, phi_w3, phi_b3,
           gcn_w1, gcn_b1, gcn_w2, gcn_b2):
  nf = jnp.stack([x[:, 0], x[:, 2], x[:, 3], x[:, 4]], axis=1)
  t = x[:, 1].astype(jnp.int32)
  emb = jnp.where((t == 0)[:, None], emb_table[0][None, :],
                  emb_table[1][None, :])
  featp = _pad(jnp.concatenate([nf, emb], axis=1), (_NP, 16))

  rowg = edge_index[0].reshape(_ER, _EC)
  colg = edge_index[1].reshape(_ER, _EC)
  ewg = edge_attr[:, 0].reshape(_ER, _EC)
  zrow = jnp.zeros((_TPB,), _f32)
  zblk = jnp.zeros((_TPB, 144), _f32)

  w0c = _pad(phi_w0[:6], (16, _HID))
  w0r = _pad(phi_w0[6:12], (16, _HID))
  w0e = phi_w0[12].reshape(1, _HID)
  b0 = phi_b0.reshape(1, _HID)
  b1 = phi_b1.reshape(1, _HID)
  b2 = phi_b2.reshape(1, _HID)
  bf16 = jnp.bfloat16
  w1b = phi_w1.astype(bf16)
  w2b = phi_w2.astype(bf16)
  w3lo, w3hi = phi_w3[:, :144].astype(bf16), phi_w3[:, 144:].astype(bf16)
  b3lo, b3hi = phi_b3[:144].reshape(1, 144), phi_b3[144:].reshape(1, 144)
  w1lo = [gcn_w1[:144, q * 48:(q + 1) * 48] for q in range(4)]
  w1hi = [gcn_w1[144:, q * 48:(q + 1) * 48] for q in range(4)]
  gb1 = [gcn_b1[q * 48:(q + 1) * 48].reshape(1, 48) for q in range(4)]
  w2qlo = [gcn_w2[q * 48:(q + 1) * 48, :64] for q in range(4)]
  w2qhi = [gcn_w2[q * 48:(q + 1) * 48, 64:] for q in range(4)]
  gb2 = gcn_b2.reshape(1, 128)
  eye64 = jnp.eye(64, dtype=_f32)
  s0 = jnp.concatenate([eye64, jnp.zeros((64, 64), _f32)], axis=1)
  s1 = jnp.concatenate([jnp.zeros((64, 64), _f32), eye64], axis=1)

  sc1 = pl.kernel(
      _sc_gather_body,
      out_type=[jax.ShapeDtypeStruct((_E, 16), _f32),
                jax.ShapeDtypeStruct((_E, 16), _f32),
                jax.ShapeDtypeStruct((2, _NP), _f32)],
      mesh=_sc_mesh(),
      compiler_params=_SC_PARAMS,
      scratch_types=[pltpu.VMEM((_EC,), jnp.int32),
                     pltpu.VMEM((_EC,), jnp.int32),
                     pltpu.VMEM((_EC, 16), _f32),
                     pltpu.VMEM((_EC, 16), _f32),
                     pltpu.VMEM((_EC,), _f32),
                     pltpu.VMEM((_TPB,), _f32),
                     pltpu.VMEM_SHARED((_NP,), _f32),
                     pltpu.SemaphoreType.DMA],
  )
  c16, r16, degp = sc1(featp, colg, rowg, ewg, zrow)

  dinv = pl.pallas_call(
      _dinv_body,
      grid=(1,),
      in_specs=[_full((2, 80, 128))],
      out_specs=_full((80, 128)),
      out_shape=jax.ShapeDtypeStruct((80, 128), _f32),
  )(degp.reshape(2, 80, 128))
  dinvc = dinv.reshape(_NP, 1)

  ne = _E // _TE
  espec = pl.BlockSpec((_TE, 16), lambda i: (i, 0))
  mspec = pl.BlockSpec((_TE, 144), lambda i: (i, 0))
  mlo, mhi = pl.pallas_call(
      _phi_body,
      grid=(ne,),
      in_specs=[espec, espec, pl.BlockSpec((_TE, 1), lambda i: (i, 0)),
                _full((16, _HID)), _full((16, _HID)), _full((1, _HID)),
                _full((1, _HID)), _full((_HID, _HID)), _full((1, _HID)),
                _full((_HID, _HID)), _full((1, _HID)),
                _full((_HID, 144)), _full((_HID, 144)),
                _full((1, 144)), _full((1, 144))],
      out_specs=[mspec, mspec],
      out_shape=[jax.ShapeDtypeStruct((_E, 144), _f32),
                 jax.ShapeDtypeStruct((_E, 144), _f32)],
  )(c16, r16, edge_attr, w0c, w0r, w0e, b0, w1b, b1, w2b, b2,
    w3lo, w3hi, b3lo, b3hi)

  sc2 = pl.kernel(
      _sc_msg_body,
      out_type=[jax.ShapeDtypeStruct((_NP, 144), _f32),
                jax.ShapeDtypeStruct((_NP, 144), _f32)],
      mesh=_sc_mesh(),
      compiler_params=_SC_PARAMS,
      scratch_types=[pltpu.VMEM((_EC,), jnp.int32),
                     pltpu.VMEM((_EC, 144), _f32),
                     pltpu.VMEM((_NHA // _NS, 144), _f32),
                     pltpu.VMEM_SHARED((_NHA, 144), _f32),
                     pltpu.SemaphoreType.DMA],
  )
  h0lo, h0hi = sc2(mlo, mhi, colg, zblk)

  nn = _NP // _TE
  nspec = lambda w: pl.BlockSpec((_TE, w), lambda i: (i, 0))
  g1 = pl.pallas_call(
      _g1_body,
      grid=(nn,),
      in_specs=[nspec(144), nspec(144), nspec(1)]
      + [_full((144, 48))] * 8,
      out_specs=[nspec(48)] * 4,
      out_shape=[jax.ShapeDtypeStruct((_NP, 48), _f32)] * 4,
  )(h0lo, h0hi, dinvc, *w1lo, *w1hi)

  def spmm(dh, glo, ghi):
    return pl.kernel(
        functools.partial(_sc_spmm_body, dh),
        out_type=[jax.ShapeDtypeStruct((_NP, dh), _f32),
                  jax.ShapeDtypeStruct((_NP, dh), _f32)],
        mesh=_sc_mesh(),
        compiler_params=_SC_PARAMS,
        scratch_types=[pltpu.VMEM((_EC,), jnp.int32),
                       pltpu.VMEM((_EC,), jnp.int32),
                       pltpu.VMEM((_EC, dh), _f32),
                       pltpu.VMEM((_EC,), _f32),
                       pltpu.VMEM((_TPB, dh), _f32),
                       pltpu.VMEM_SHARED((_NP, dh), _f32),
                       pltpu.SemaphoreType.DMA],
    )(glo, ghi, colg, rowg, ewg, zblk)

  t1a, t1c = spmm(48, g1[0], g1[2])
  t1b, t1d = spmm(48, g1[1], g1[3])

  g2lo, g2hi = pl.pallas_call(
      _g2_body,
      grid=(nn,),
      in_specs=[nspec(48)] * 4 + [nspec(1)] + [_full((1, 48))] * 4
      + [_full((48, 64))] * 8,
      out_specs=[nspec(64), nspec(64)],
      out_shape=[jax.ShapeDtypeStruct((_NP, 64), _f32),
                 jax.ShapeDtypeStruct((_NP, 64), _f32)],
  )(t1a, t1b, t1c, t1d, dinvc, *gb1, *w2qlo, *w2qhi)

  t2lo, t2hi = spmm(64, g2lo, g2hi)

  out = pl.pallas_call(
      _out_body,
      grid=(nn,),
      in_specs=[nspec(64), nspec(64), nspec(1),
                _full((64, 128)), _full((64, 128)), _full((1, 128))],
      out_specs=nspec(128),
      out_shape=jax.ShapeDtypeStruct((_NP, 128), _f32),
  )(t2lo, t2hi, dinvc, s0, s1, gb2)

  return out[:_N]


# speedup vs baseline: 1.1397x; 1.1397x over previous
import functools

import jax
import jax.numpy as jnp
from jax import lax
from jax.experimental import pallas as pl
from jax.experimental.pallas import tpu as pltpu
from jax.experimental.pallas import tpu_sc as plsc

_N = 10000
_NP = 10240
_NH = 5120
_NHA = 5136
_E = 160000
_ER = 1250
_EC = 128
_HID = 1024
_NS = 16
_TPB = _NP // _NS
_TE = 640

_f32 = jnp.float32
_bf16 = jnp.bfloat16
_SC_PARAMS = pltpu.CompilerParams(use_tc_tiling_on_sc=False,
                                  needs_layout_passes=False)


def _sc_mesh():
  return plsc.VectorSubcoreMesh(core_axis_name="c", subcore_axis_name="s")


def _sc_gather_body(feat_hbm, colg_hbm, rowg_hbm, ewg_hbm, zrow_hbm,
                    c16_hbm, r16_hbm, degp_hbm, cntp_hbm,
                    idxc_v, idxr_v, crows_v, rrows_v, ew_v, ones_v, z_v,
                    dacc_sh, cacc_sh, sem):
  cid = lax.axis_index("c")
  sid = lax.axis_index("s")
  def mkones(k, carry):
    ones_v[pl.ds(k * 16, 16)] = jnp.zeros((16,), _f32) + 1.0
    return carry
  lax.fori_loop(0, _EC // 16, mkones, 0)
  pltpu.sync_copy(zrow_hbm, z_v)
  pltpu.sync_copy(z_v, dacc_sh.at[pl.ds(sid * _TPB, _TPB)])
  pltpu.sync_copy(z_v, cacc_sh.at[pl.ds(sid * _TPB, _TPB)])
  plsc.subcore_barrier()

  def body(g, carry):
    lr = sid + g * _NS

    @pl.when(lr < _ER // 2)
    def _():
      r = cid * (_ER // 2) + lr
      pltpu.sync_copy(colg_hbm.at[r], idxc_v)
      pltpu.sync_copy(rowg_hbm.at[r], idxr_v)
      pltpu.async_copy(feat_hbm.at[idxc_v], crows_v, sem).wait()
      pltpu.sync_copy(crows_v, c16_hbm.at[pl.ds(r * _EC, _EC)])
      pltpu.async_copy(feat_hbm.at[idxr_v], rrows_v, sem).wait()
      pltpu.sync_copy(rrows_v, r16_hbm.at[pl.ds(r * _EC, _EC)])
      pltpu.sync_copy(ewg_hbm.at[r], ew_v)
      pltpu.sync_copy(ew_v, dacc_sh.at[idxc_v], add=True)
      pltpu.sync_copy(ones_v, cacc_sh.at[idxc_v], add=True)
    return carry

  lax.fori_loop(0, (_ER // 2 + _NS - 1) // _NS, body, 0)
  plsc.subcore_barrier()
  pltpu.sync_copy(dacc_sh.at[pl.ds(sid * _TPB, _TPB)],
                  degp_hbm.at[cid, pl.ds(sid * _TPB, _TPB)])
  pltpu.sync_copy(cacc_sh.at[pl.ds(sid * _TPB, _TPB)],
                  cntp_hbm.at[cid, pl.ds(sid * _TPB, _TPB)])


def _sc_msg_body(ma_hbm, mb_hbm, colg_hbm, zblk_hbm,
                 h0a_hbm, h0b_hbm,
                 idx_v, mrow_v, z_v, acc_sh, sem):
  cid = lax.axis_index("c")
  sid = lax.axis_index("s")
  rows_per_tile = _NHA // _NS
  pltpu.sync_copy(zblk_hbm.at[pl.ds(0, rows_per_tile)], z_v)

  for p in range(2):
    base = p * _NH
    pltpu.sync_copy(z_v, acc_sh.at[pl.ds(sid * rows_per_tile, rows_per_tile)])
    plsc.subcore_barrier()

    def body(g, carry):
      r = sid + g * _NS

      @pl.when(r < _ER)
      def _():
        pltpu.sync_copy(colg_hbm.at[r], idx_v)
        for k in range(_EC // 16):
          v = idx_v[pl.ds(k * 16, 16)] - base
          ok = jnp.logical_and(v >= 0, v < _NH)
          idx_v[pl.ds(k * 16, 16)] = jnp.where(ok, v, _NH)

        @pl.when(cid == 0)
        def _():
          pltpu.sync_copy(ma_hbm.at[pl.ds(r * _EC, _EC)], mrow_v)

        @pl.when(cid == 1)
        def _():
          pltpu.sync_copy(mb_hbm.at[pl.ds(r * _EC, _EC)], mrow_v)

        pltpu.sync_copy(mrow_v, acc_sh.at[idx_v], add=True)
      return carry

    lax.fori_loop(0, (_ER + _NS - 1) // _NS, body, 0)
    plsc.subcore_barrier()
    out_rows = _NH // _NS
    src = pl.ds(sid * out_rows, out_rows)
    dst = pl.ds(base + sid * out_rows, out_rows)

    @pl.when(cid == 0)
    def _():
      pltpu.sync_copy(acc_sh.at[src], h0a_hbm.at[dst])

    @pl.when(cid == 1)
    def _():
      pltpu.sync_copy(acc_sh.at[src, pl.ds(0, 64)], h0b_hbm.at[dst])

    plsc.subcore_barrier()


def _sc_spmm_body(dh, glo_hbm, ghi_hbm, colg_hbm, rowg_hbm, ewg_hbm,
                  zblk_hbm, tlo_hbm, thi_hbm,
                  idxc_v, idxr_v, grow_v, ew_v, z_v, acc_sh, sem):
  cid = lax.axis_index("c")
  sid = lax.axis_index("s")
  pltpu.sync_copy(zblk_hbm.at[:, pl.ds(0, dh)], z_v)
  pltpu.sync_copy(z_v, acc_sh.at[pl.ds(sid * _TPB, _TPB)])
  plsc.subcore_barrier()

  def body(g, carry):
    r = sid + g * _NS

    @pl.when(r < _ER)
    def _():
      pltpu.sync_copy(colg_hbm.at[r], idxc_v)
      pltpu.sync_copy(rowg_hbm.at[r], idxr_v)
      pltpu.sync_copy(ewg_hbm.at[r], ew_v)

      @pl.when(cid == 0)
      def _():
        pltpu.async_copy(glo_hbm.at[idxr_v], grow_v, sem).wait()

      @pl.when(cid == 1)
      def _():
        pltpu.async_copy(ghi_hbm.at[idxr_v], grow_v, sem).wait()

      def escale(e, c2):
        w = plsc.load_gather(ew_v, [jnp.zeros((16,), jnp.int32) + e])
        for j in range(dh // 16):
          grow_v[e, pl.ds(j * 16, 16)] = grow_v[e, pl.ds(j * 16, 16)] * w
        return c2

      lax.fori_loop(0, _EC, escale, 0)
      pltpu.sync_copy(grow_v, acc_sh.at[idxc_v], add=True)
    return carry

  lax.fori_loop(0, (_ER + _NS - 1) // _NS, body, 0)
  plsc.subcore_barrier()
  sl = pl.ds(sid * _TPB, _TPB)

  @pl.when(cid == 0)
  def _():
    pltpu.sync_copy(acc_sh.at[sl], tlo_hbm.at[sl])

  @pl.when(cid == 1)
  def _():
    pltpu.sync_copy(acc_sh.at[sl], thi_hbm.at[sl])


def _prep_body(degp_ref, cntp_ref, w3_ref, gw1_ref, b3_ref,
               dinv_ref, cnt_ref, w3w1_ref, c1_ref):
  deg = degp_ref[0] + degp_ref[1]
  dinv_ref[...] = jnp.where(deg > 0, lax.rsqrt(jnp.maximum(deg, 1e-30)), 0.0)
  cnt_ref[...] = cntp_ref[0] + cntp_ref[1]
  w3w1_ref[...] = jnp.dot(w3_ref[...], gw1_ref[...],
                          preferred_element_type=_f32)
  c1_ref[...] = jnp.dot(b3_ref[...], gw1_ref[...],
                        preferred_element_type=_f32)


def _phi_body(c16_ref, r16_ref, ew_ref, w0c_ref, w0r_ref, w0e_ref, b0_ref,
              w1_ref, b1_ref, w2_ref, b2_ref, wpa_ref, wpb_ref,
              ma_ref, mb_ref):
  h = jnp.dot(c16_ref[...], w0c_ref[...], preferred_element_type=_f32)
  h = h + jnp.dot(r16_ref[...], w0r_ref[...], preferred_element_type=_f32)
  h = h + ew_ref[...] * w0e_ref[...]
  h = jax.nn.relu(h + b0_ref[...]).astype(_bf16)
  h = jax.nn.relu(jnp.dot(h, w1_ref[...], preferred_element_type=_f32)
                  + b1_ref[...]).astype(_bf16)
  h = jax.nn.relu(jnp.dot(h, w2_ref[...], preferred_element_type=_f32)
                  + b2_ref[...]).astype(_bf16)
  ma_ref[...] = jnp.dot(h, wpa_ref[...], preferred_element_type=_f32)
  mb_ref[...] = jnp.dot(h, wpb_ref[...], preferred_element_type=_f32)


def _g1_body(h0a_ref, h0b_ref, dinv_ref, cnt_ref, c1a_ref, c1b_ref,
             ta0_ref, ta1_ref, tb0_ref, tb1_ref):
  d = dinv_ref[...]
  cnt = cnt_ref[...]
  va = d * (h0a_ref[...] + cnt * c1a_ref[...])
  vb = d * (h0b_ref[...] + cnt * c1b_ref[...])
  ta0_ref[...] = va[:, :64]
  ta1_ref[...] = va[:, 64:]
  tb0_ref[...] = vb[:, :32]
  tb1_ref[...] = vb[:, 32:]


def _g2_body(u0_ref, u1_ref, u2_ref, u3_ref, dinv_ref,
             b1a_ref, b1b_ref, b1c_ref, b1d_ref,
             wa_lo, wb_lo, wc_lo, wd_lo, wa_hi, wb_hi, wc_hi, wd_hi,
             glo_ref, ghi_ref):
  d = dinv_ref[...]
  h1 = [jax.nn.relu(d * t[...] + b[...])
        for t, b in ((u0_ref, b1a_ref), (u1_ref, b1b_ref),
                     (u2_ref, b1c_ref), (u3_ref, b1d_ref))]
  wlo = (wa_lo, wb_lo, wc_lo, wd_lo)
  whi = (wa_hi, wb_hi, wc_hi, wd_hi)
  glo = jnp.dot(h1[0], wlo[0][...], preferred_element_type=_f32)
  ghi = jnp.dot(h1[0], whi[0][...], preferred_element_type=_f32)
  for q in range(1, 4):
    glo = glo + jnp.dot(h1[q], wlo[q][...], preferred_element_type=_f32)
    ghi = ghi + jnp.dot(h1[q], whi[q][...], preferred_element_type=_f32)
  glo_ref[...] = d * glo
  ghi_ref[...] = d * ghi


def _out_body(t2lo_ref, t2hi_ref, dinv_ref, s0_ref, s1_ref, b2_ref, out_ref):
  d = dinv_ref[...]
  t2 = jnp.dot(t2lo_ref[...], s0_ref[...], preferred_element_type=_f32) \
      + jnp.dot(t2hi_ref[...], s1_ref[...], preferred_element_type=_f32)
  out_ref[...] = jax.nn.relu(d * t2 + b2_ref[...])


def _full(shape):
  return pl.BlockSpec(shape, lambda *_: tuple(0 for _ in shape))


def _pad(a, shape):
  out = jnp.zeros(shape, a.dtype)
  return out.at[tuple(slice(0, s) for s in a.shape)].set(a)


def kernel(x, edge_attr, edge_index, emb_table,
           phi_w0, phi_b0, phi_w1, phi_b1, phi_w2, phi_b2, phi_w3, phi_b3,
           gcn_w1, gcn_b1, gcn_w2, gcn_b2):
  nf = jnp.stack([x[:, 0], x[:, 2], x[:, 3], x[:, 4]], axis=1)
  t = x[:, 1].astype(jnp.int32)
  emb = jnp.where((t == 0)[:, None], emb_table[0][None, :],
                  emb_table[1][None, :])
  featp = _pad(jnp.concatenate([nf, emb], axis=1), (_NP, 16))

  rowg = edge_index[0].reshape(_ER, _EC)
  colg = edge_index[1].reshape(_ER, _EC)
  ewg = edge_attr[:, 0].reshape(_ER, _EC)
  zrow = jnp.zeros((_TPB,), _f32)
  zblk = jnp.zeros((_TPB, 128), _f32)

  w0c = _pad(phi_w0[:6], (16, _HID))
  w0r = _pad(phi_w0[6:12], (16, _HID))
  w0e = phi_w0[12].reshape(1, _HID)
  b0 = phi_b0.reshape(1, _HID)
  b1 = phi_b1.reshape(1, _HID)
  b2 = phi_b2.reshape(1, _HID)
  w1b = phi_w1.astype(_bf16)
  w2b = phi_w2.astype(_bf16)
  gb1 = [gcn_b1[0:64].reshape(1, 64), gcn_b1[64:128].reshape(1, 64),
         gcn_b1[128:160].reshape(1, 32), gcn_b1[160:192].reshape(1, 32)]
  w2rows = [gcn_w2[0:64], gcn_w2[64:128], gcn_w2[128:160], gcn_w2[160:192]]
  w2qlo = [w[:, :64] for w in w2rows]
  w2qhi = [w[:, 64:] for w in w2rows]
  gb2 = gcn_b2.reshape(1, 128)
  eye64 = jnp.eye(64, dtype=_f32)
  s0 = jnp.concatenate([eye64, jnp.zeros((64, 64), _f32)], axis=1)
  s1 = jnp.concatenate([jnp.zeros((64, 64), _f32), eye64], axis=1)

  sc1 = pl.kernel(
      _sc_gather_body,
      out_type=[jax.ShapeDtypeStruct((_E, 16), _f32),
                jax.ShapeDtypeStruct((_E, 16), _f32),
                jax.ShapeDtypeStruct((2, _NP), _f32),
                jax.ShapeDtypeStruct((2, _NP), _f32)],
      mesh=_sc_mesh(),
      compiler_params=_SC_PARAMS,
      scratch_types=[pltpu.VMEM((_EC,), jnp.int32),
                     pltpu.VMEM((_EC,), jnp.int32),
                     pltpu.VMEM((_EC, 16), _f32),
                     pltpu.VMEM((_EC, 16), _f32),
                     pltpu.VMEM((_EC,), _f32),
                     pltpu.VMEM((_EC,), _f32),
                     pltpu.VMEM((_TPB,), _f32),
                     pltpu.VMEM_SHARED((_NP,), _f32),
                     pltpu.VMEM_SHARED((_NP,), _f32),
                     pltpu.SemaphoreType.DMA],
  )
  c16, r16, degp, cntp = sc1(featp, colg, rowg, ewg, zrow)

  dinv, cnt, w3w1, c1 = pl.pallas_call(
      _prep_body,
      grid=(1,),
      in_specs=[_full((2, 80, 128)), _full((2, 80, 128)),
                _full((_HID, 288)), _full((288, 192)), _full((1, 288))],
      out_specs=[_full((80, 128)), _full((80, 128)),
                 _full((_HID, 192)), _full((1, 192))],
      out_shape=[jax.ShapeDtypeStruct((80, 128), _f32),
                 jax.ShapeDtypeStruct((80, 128), _f32),
                 jax.ShapeDtypeStruct((_HID, 192), _f32),
                 jax.ShapeDtypeStruct((1, 192), _f32)],
  )(degp.reshape(2, 80, 128), cntp.reshape(2, 80, 128), phi_w3, gcn_w1,
    phi_b3.reshape(1, 288))
  dinvc = dinv.reshape(_NP, 1)
  cntc = cnt.reshape(_NP, 1)
  wpa = w3w1[:, :128].astype(_bf16)
  wpb = _pad(w3w1[:, 128:].astype(_bf16), (_HID, 128))
  c1a = c1[:, :128]
  c1b = c1[:, 128:]

  ne = _E // _TE
  espec = pl.BlockSpec((_TE, 16), lambda i: (i, 0))
  mspec = pl.BlockSpec((_TE, 128), lambda i: (i, 0))
  ma, mb = pl.pallas_call(
      _phi_body,
      grid=(ne,),
      in_specs=[espec, espec, pl.BlockSpec((_TE, 1), lambda i: (i, 0)),
                _full((16, _HID)), _full((16, _HID)), _full((1, _HID)),
                _full((1, _HID)), _full((_HID, _HID)), _full((1, _HID)),
                _full((_HID, _HID)), _full((1, _HID)),
                _full((_HID, 128)), _full((_HID, 128))],
      out_specs=[mspec, mspec],
      out_shape=[jax.ShapeDtypeStruct((_E, 128), _f32),
                 jax.ShapeDtypeStruct((_E, 128), _f32)],
  )(c16, r16, edge_attr, w0c, w0r, w0e, b0, w1b, b1, w2b, b2, wpa, wpb)

  sc2 = pl.kernel(
      _sc_msg_body,
      out_type=[jax.ShapeDtypeStruct((_NP, 128), _f32),
                jax.ShapeDtypeStruct((_NP, 64), _f32)],
      mesh=_sc_mesh(),
      compiler_params=_SC_PARAMS,
      scratch_types=[pltpu.VMEM((_EC,), jnp.int32),
                     pltpu.VMEM((_EC, 128), _f32),
                     pltpu.VMEM((_NHA // _NS, 128), _f32),
                     pltpu.VMEM_SHARED((_NHA, 128), _f32),
                     pltpu.SemaphoreType.DMA],
  )
  h0a, h0b = sc2(ma, mb, colg, zblk)

  nn = _NP // _TE
  nspec = lambda w: pl.BlockSpec((_TE, w), lambda i: (i, 0))
  ta0, ta1, tb0, tb1 = pl.pallas_call(
      _g1_body,
      grid=(nn,),
      in_specs=[nspec(128), nspec(64), nspec(1), nspec(1),
                _full((1, 128)), _full((1, 64))],
      out_specs=[nspec(64), nspec(64), nspec(32), nspec(32)],
      out_shape=[jax.ShapeDtypeStruct((_NP, 64), _f32),
                 jax.ShapeDtypeStruct((_NP, 64), _f32),
                 jax.ShapeDtypeStruct((_NP, 32), _f32),
                 jax.ShapeDtypeStruct((_NP, 32), _f32)],
  )(h0a, h0b, dinvc, cntc, c1a, c1b)

  def spmm(dh, glo, ghi):
    return pl.kernel(
        functools.partial(_sc_spmm_body, dh),
        out_type=[jax.ShapeDtypeStruct((_NP, dh), _f32),
                  jax.ShapeDtypeStruct((_NP, dh), _f32)],
        mesh=_sc_mesh(),
        compiler_params=_SC_PARAMS,
        scratch_types=[pltpu.VMEM((_EC,), jnp.int32),
                       pltpu.VMEM((_EC,), jnp.int32),
                       pltpu.VMEM((_EC, dh), _f32),
                       pltpu.VMEM((_EC,), _f32),
                       pltpu.VMEM((_TPB, dh), _f32),
                       pltpu.VMEM_SHARED((_NP, dh), _f32),
                       pltpu.SemaphoreType.DMA],
    )(glo, ghi, colg, rowg, ewg, zblk)

  u0, u1 = spmm(64, ta0, ta1)
  u2, u3 = spmm(32, tb0, tb1)

  g2lo, g2hi = pl.pallas_call(
      _g2_body,
      grid=(nn,),
      in_specs=[nspec(64), nspec(64), nspec(32), nspec(32), nspec(1),
                _full((1, 64)), _full((1, 64)), _full((1, 32)),
                _full((1, 32)),
                _full((64, 64)), _full((64, 64)), _full((32, 64)),
                _full((32, 64)),
                _full((64, 64)), _full((64, 64)), _full((32, 64)),
                _full((32, 64))],
      out_specs=[nspec(64), nspec(64)],
      out_shape=[jax.ShapeDtypeStruct((_NP, 64), _f32),
                 jax.ShapeDtypeStruct((_NP, 64), _f32)],
  )(u0, u1, u2, u3, dinvc, *gb1, *w2qlo, *w2qhi)

  t2lo, t2hi = spmm(64, g2lo, g2hi)

  out = pl.pallas_call(
      _out_body,
      grid=(nn,),
      in_specs=[nspec(64), nspec(64), nspec(1),
                _full((64, 128)), _full((64, 128)), _full((1, 128))],
      out_specs=nspec(128),
      out_shape=jax.ShapeDtypeStruct((_NP, 128), _f32),
  )(t2lo, t2hi, dinvc, s0, s1, gb2)

  return out[:_N]


# speedup vs baseline: 1.2607x; 1.1062x over previous
import functools

import jax
import jax.numpy as jnp
from jax import lax
from jax.experimental import pallas as pl
from jax.experimental.pallas import tpu as pltpu
from jax.experimental.pallas import tpu_sc as plsc

_N = 10000
_NP = 10240
_NH = 5120
_NHA = 5136
_E = 160000
_ER = 1250
_EC = 128
_HID = 1024
_NS = 16
_TPB = _NP // _NS
_TE = 640

_f32 = jnp.float32
_bf16 = jnp.bfloat16
_SC_PARAMS = pltpu.CompilerParams(use_tc_tiling_on_sc=False,
                                  needs_layout_passes=False)


def _sc_mesh():
  return plsc.VectorSubcoreMesh(core_axis_name="c", subcore_axis_name="s")


def _sc_gather_body(feat_hbm, colg_hbm, rowg_hbm, ewg_hbm, zrow_hbm,
                    c16_hbm, r16_hbm, degp_hbm, cntp_hbm,
                    idxc_v, idxr_v, crows_v, rrows_v, ew_v, ones_v, z_v,
                    dacc_sh, cacc_sh, sem):
  cid = lax.axis_index("c")
  sid = lax.axis_index("s")
  def mkones(k, carry):
    ones_v[pl.ds(k * 16, 16)] = jnp.zeros((16,), _f32) + 1.0
    return carry
  lax.fori_loop(0, _EC // 16, mkones, 0)
  pltpu.sync_copy(zrow_hbm, z_v)
  pltpu.sync_copy(z_v, dacc_sh.at[pl.ds(sid * _TPB, _TPB)])
  pltpu.sync_copy(z_v, cacc_sh.at[pl.ds(sid * _TPB, _TPB)])
  plsc.subcore_barrier()

  def body(g, carry):
    lr = sid + g * _NS

    @pl.when(lr < _ER // 2)
    def _():
      r = cid * (_ER // 2) + lr
      pltpu.sync_copy(colg_hbm.at[r], idxc_v)
      pltpu.sync_copy(rowg_hbm.at[r], idxr_v)
      pltpu.async_copy(feat_hbm.at[idxc_v], crows_v, sem).wait()
      pltpu.sync_copy(crows_v, c16_hbm.at[pl.ds(r * _EC, _EC)])
      pltpu.async_copy(feat_hbm.at[idxr_v], rrows_v, sem).wait()
      pltpu.sync_copy(rrows_v, r16_hbm.at[pl.ds(r * _EC, _EC)])
      pltpu.sync_copy(ewg_hbm.at[r], ew_v)
      pltpu.sync_copy(ew_v, dacc_sh.at[idxc_v], add=True)
      pltpu.sync_copy(ones_v, cacc_sh.at[idxc_v], add=True)
    return carry

  lax.fori_loop(0, (_ER // 2 + _NS - 1) // _NS, body, 0)
  plsc.subcore_barrier()
  pltpu.sync_copy(dacc_sh.at[pl.ds(sid * _TPB, _TPB)],
                  degp_hbm.at[cid, pl.ds(sid * _TPB, _TPB)])
  pltpu.sync_copy(cacc_sh.at[pl.ds(sid * _TPB, _TPB)],
                  cntp_hbm.at[cid, pl.ds(sid * _TPB, _TPB)])


def _sc_msg_body(ma_hbm, mb_hbm, colg_hbm, zblk_hbm,
                 h0a_hbm, h0b_hbm,
                 idx_v, mr0_v, mr1_v, mr2_v, mr3_v, z_v, acc_sh, sem):
  cid = lax.axis_index("c")
  sid = lax.axis_index("s")
  rows_per_tile = _NHA // _NS
  pltpu.sync_copy(zblk_hbm.at[pl.ds(0, rows_per_tile // 3)], z_v)

  for p in range(2):
    base = p * _NH
    for tz in range(3):
      pltpu.sync_copy(
          z_v, acc_sh.at[pl.ds(sid * rows_per_tile + tz * 107, 107)])
    plsc.subcore_barrier()

    def remap(j, nk):
      for k in range(nk):
        v = idx_v[j, pl.ds(k * 16, 16)] - base
        ok = jnp.logical_and(v >= 0, v < _NH)
        idx_v[j, pl.ds(k * 16, 16)] = jnp.where(ok, v, _NH)

    ngrp = _ER // 4

    def body(g2, carry):
      g = sid + g2 * _NS

      @pl.when(g < ngrp)
      def _():
        pltpu.sync_copy(colg_hbm.at[pl.ds(g * 4, 4)], idx_v)
        for j in range(4):
          remap(j, _EC // 16)

        mrows = (mr0_v, mr1_v, mr2_v, mr3_v)
        for j in range(4):

          @pl.when(cid == 0)
          def _():
            pltpu.sync_copy(ma_hbm.at[pl.ds((g * 4 + j) * _EC, _EC)],
                            mrows[j])

          @pl.when(cid == 1)
          def _():
            pltpu.sync_copy(mb_hbm.at[pl.ds((g * 4 + j) * _EC, _EC)],
                            mrows[j])

        for j in range(4):
          pltpu.sync_copy(mrows[j], acc_sh.at[idx_v.at[j]], add=True)
      return carry

    lax.fori_loop(0, (ngrp + _NS - 1) // _NS, body, 0)

    @pl.when(sid < _ER - 4 * ngrp)
    def _():
      r = 4 * ngrp + sid
      pltpu.sync_copy(colg_hbm.at[r], idx_v.at[0])
      remap(0, _EC // 16)

      @pl.when(cid == 0)
      def _():
        pltpu.sync_copy(ma_hbm.at[pl.ds(r * _EC, _EC)], mr0_v)

      @pl.when(cid == 1)
      def _():
        pltpu.sync_copy(mb_hbm.at[pl.ds(r * _EC, _EC)], mr0_v)

      pltpu.sync_copy(mr0_v, acc_sh.at[idx_v.at[0]], add=True)

    plsc.subcore_barrier()
    out_rows = _NH // _NS
    src = pl.ds(sid * out_rows, out_rows)
    dst = pl.ds(base + sid * out_rows, out_rows)

    @pl.when(cid == 0)
    def _():
      pltpu.sync_copy(acc_sh.at[src], h0a_hbm.at[dst])

    @pl.when(cid == 1)
    def _():
      pltpu.sync_copy(acc_sh.at[src, pl.ds(0, 64)], h0b_hbm.at[dst])

    plsc.subcore_barrier()


def _sc_spmm_body(dh, glo_hbm, ghi_hbm, colg_hbm, rowg_hbm, ewf_hbm,
                  zblk_hbm, tlo_hbm, thi_hbm,
                  idxc_v, idxr_v, gr0_v, gr1_v, gr2_v, gr3_v, ew_v, z_v,
                  acc_sh, sem):
  cid = lax.axis_index("c")
  sid = lax.axis_index("s")
  pltpu.sync_copy(zblk_hbm.at[pl.ds(0, _TPB // 4), pl.ds(0, dh)], z_v)
  for tz in range(4):
    pltpu.sync_copy(z_v, acc_sh.at[pl.ds(sid * _TPB + tz * (_TPB // 4),
                                         _TPB // 4)])
  plsc.subcore_barrier()

  grows = (gr0_v, gr1_v, gr2_v, gr3_v)

  def escale(j, ebase):
    buf = grows[j]
    def one(e, c2):
      w = plsc.load_gather(ew_v, [jnp.zeros((16,), jnp.int32) + ebase + e])
      for q in range(dh // 16):
        buf[e, pl.ds(q * 16, 16)] = buf[e, pl.ds(q * 16, 16)] * w
      return c2
    lax.fori_loop(0, _EC, one, 0)

  def gather_rows(j):

    @pl.when(cid == 0)
    def _():
      pltpu.async_copy(glo_hbm.at[idxr_v.at[j]], grows[j], sem).wait()

    @pl.when(cid == 1)
    def _():
      pltpu.async_copy(ghi_hbm.at[idxr_v.at[j]], grows[j], sem).wait()

  ngrp = _ER // 4
  def body(k, carry):
    g = sid + k * _NS

    @pl.when(g < ngrp)
    def _():
      pltpu.sync_copy(colg_hbm.at[pl.ds(g * 4, 4)], idxc_v)
      pltpu.sync_copy(rowg_hbm.at[pl.ds(g * 4, 4)], idxr_v)
      pltpu.sync_copy(ewf_hbm.at[pl.ds(g * 4 * _EC, 4 * _EC)], ew_v)
      for j in range(4):
        gather_rows(j)
      for j in range(4):
        escale(j, j * _EC)
      for j in range(4):
        pltpu.sync_copy(grows[j], acc_sh.at[idxc_v.at[j]], add=True)
    return carry

  lax.fori_loop(0, (ngrp + _NS - 1) // _NS, body, 0)

  @pl.when(sid < _ER - 4 * ngrp)
  def _():
    r = 4 * ngrp + sid
    pltpu.sync_copy(colg_hbm.at[r], idxc_v.at[0])
    pltpu.sync_copy(rowg_hbm.at[r], idxr_v.at[0])
    pltpu.sync_copy(ewf_hbm.at[pl.ds(r * _EC, _EC)], ew_v.at[pl.ds(0, _EC)])
    gather_rows(0)
    escale(0, 0)
    pltpu.sync_copy(gr0_v, acc_sh.at[idxc_v.at[0]], add=True)

  plsc.subcore_barrier()
  sl = pl.ds(sid * _TPB, _TPB)

  @pl.when(cid == 0)
  def _():
    pltpu.sync_copy(acc_sh.at[sl], tlo_hbm.at[sl])

  @pl.when(cid == 1)
  def _():
    pltpu.sync_copy(acc_sh.at[sl], thi_hbm.at[sl])


def _prep_body(degp_ref, cntp_ref, w3_ref, gw1_ref, b3_ref,
               dinv_ref, cnt_ref, w3w1_ref, c1_ref):
  deg = degp_ref[0] + degp_ref[1]
  dinv_ref[...] = jnp.where(deg > 0, lax.rsqrt(jnp.maximum(deg, 1e-30)), 0.0)
  cnt_ref[...] = cntp_ref[0] + cntp_ref[1]
  w3w1_ref[...] = jnp.dot(w3_ref[...], gw1_ref[...],
                          preferred_element_type=_f32)
  c1_ref[...] = jnp.dot(b3_ref[...], gw1_ref[...],
                        preferred_element_type=_f32)


def _phi_body(c16_ref, r16_ref, ew_ref, w0c_ref, w0r_ref, w0e_ref, b0_ref,
              w1_ref, b1_ref, w2_ref, b2_ref, wpa_ref, wpb_ref,
              ma_ref, mb_ref):
  h = jnp.dot(c16_ref[...], w0c_ref[...], preferred_element_type=_f32)
  h = h + jnp.dot(r16_ref[...], w0r_ref[...], preferred_element_type=_f32)
  h = h + ew_ref[...] * w0e_ref[...]
  h = jax.nn.relu(h + b0_ref[...]).astype(_bf16)
  h = jax.nn.relu(jnp.dot(h, w1_ref[...], preferred_element_type=_f32)
                  + b1_ref[...]).astype(_bf16)
  h = jax.nn.relu(jnp.dot(h, w2_ref[...], preferred_element_type=_f32)
                  + b2_ref[...]).astype(_bf16)
  ma_ref[...] = jnp.dot(h, wpa_ref[...], preferred_element_type=_f32)
  mb_ref[...] = jnp.dot(h, wpb_ref[...], preferred_element_type=_f32)


def _g1_body(h0a_ref, h0b_ref, dinv_ref, cnt_ref, c1a_ref, c1b_ref,
             ta0_ref, ta1_ref, tb0_ref, tb1_ref):
  d = dinv_ref[...]
  cnt = cnt_ref[...]
  va = d * (h0a_ref[...] + cnt * c1a_ref[...])
  vb = d * (h0b_ref[...] + cnt * c1b_ref[...])
  ta0_ref[...] = va[:, :64]
  ta1_ref[...] = va[:, 64:]
  tb0_ref[...] = vb[:, :32]
  tb1_ref[...] = vb[:, 32:]


def _g2_body(u0_ref, u1_ref, u2_ref, u3_ref, dinv_ref,
             b1a_ref, b1b_ref, b1c_ref, b1d_ref,
             wa_lo, wb_lo, wc_lo, wd_lo, wa_hi, wb_hi, wc_hi, wd_hi,
             glo_ref, ghi_ref):
  d = dinv_ref[...]
  h1 = [jax.nn.relu(d * t[...] + b[...])
        for t, b in ((u0_ref, b1a_ref), (u1_ref, b1b_ref),
                     (u2_ref, b1c_ref), (u3_ref, b1d_ref))]
  wlo = (wa_lo, wb_lo, wc_lo, wd_lo)
  whi = (wa_hi, wb_hi, wc_hi, wd_hi)
  glo = jnp.dot(h1[0], wlo[0][...], preferred_element_type=_f32)
  ghi = jnp.dot(h1[0], whi[0][...], preferred_element_type=_f32)
  for q in range(1, 4):
    glo = glo + jnp.dot(h1[q], wlo[q][...], preferred_element_type=_f32)
    ghi = ghi + jnp.dot(h1[q], whi[q][...], preferred_element_type=_f32)
  glo_ref[...] = d * glo
  ghi_ref[...] = d * ghi


def _out_body(t2lo_ref, t2hi_ref, dinv_ref, s0_ref, s1_ref, b2_ref, out_ref):
  d = dinv_ref[...]
  t2 = jnp.dot(t2lo_ref[...], s0_ref[...], preferred_element_type=_f32) \
      + jnp.dot(t2hi_ref[...], s1_ref[...], preferred_element_type=_f32)
  out_ref[...] = jax.nn.relu(d * t2 + b2_ref[...])


def _full(shape):
  return pl.BlockSpec(shape, lambda *_: tuple(0 for _ in shape))


def _pad(a, shape):
  out = jnp.zeros(shape, a.dtype)
  return out.at[tuple(slice(0, s) for s in a.shape)].set(a)


def kernel(x, edge_attr, edge_index, emb_table,
           phi_w0, phi_b0, phi_w1, phi_b1, phi_w2, phi_b2, phi_w3, phi_b3,
           gcn_w1, gcn_b1, gcn_w2, gcn_b2):
  nf = jnp.stack([x[:, 0], x[:, 2], x[:, 3], x[:, 4]], axis=1)
  t = x[:, 1].astype(jnp.int32)
  emb = jnp.where((t == 0)[:, None], emb_table[0][None, :],
                  emb_table[1][None, :])
  featp = _pad(jnp.concatenate([nf, emb], axis=1), (_NP, 16))

  rowg = edge_index[0].reshape(_ER, _EC)
  colg = edge_index[1].reshape(_ER, _EC)
  ewf = edge_attr[:, 0]
  ewg = ewf.reshape(_ER, _EC)
  zrow = jnp.zeros((_TPB,), _f32)
  zblk = jnp.zeros((_TPB, 128), _f32)

  w0c = _pad(phi_w0[:6], (16, _HID))
  w0r = _pad(phi_w0[6:12], (16, _HID))
  w0e = phi_w0[12].reshape(1, _HID)
  b0 = phi_b0.reshape(1, _HID)
  b1 = phi_b1.reshape(1, _HID)
  b2 = phi_b2.reshape(1, _HID)
  w1b = phi_w1.astype(_bf16)
  w2b = phi_w2.astype(_bf16)
  gb1 = [gcn_b1[0:64].reshape(1, 64), gcn_b1[64:128].reshape(1, 64),
         gcn_b1[128:160].reshape(1, 32), gcn_b1[160:192].reshape(1, 32)]
  w2rows = [gcn_w2[0:64], gcn_w2[64:128], gcn_w2[128:160], gcn_w2[160:192]]
  w2qlo = [w[:, :64] for w in w2rows]
  w2qhi = [w[:, 64:] for w in w2rows]
  gb2 = gcn_b2.reshape(1, 128)
  eye64 = jnp.eye(64, dtype=_f32)
  s0 = jnp.concatenate([eye64, jnp.zeros((64, 64), _f32)], axis=1)
  s1 = jnp.concatenate([jnp.zeros((64, 64), _f32), eye64], axis=1)

  sc1 = pl.kernel(
      _sc_gather_body,
      out_type=[jax.ShapeDtypeStruct((_E, 16), _f32),
                jax.ShapeDtypeStruct((_E, 16), _f32),
                jax.ShapeDtypeStruct((2, _NP), _f32),
                jax.ShapeDtypeStruct((2, _NP), _f32)],
      mesh=_sc_mesh(),
      compiler_params=_SC_PARAMS,
      scratch_types=[pltpu.VMEM((_EC,), jnp.int32),
                     pltpu.VMEM((_EC,), jnp.int32),
                     pltpu.VMEM((_EC, 16), _f32),
                     pltpu.VMEM((_EC, 16), _f32),
                     pltpu.VMEM((_EC,), _f32),
                     pltpu.VMEM((_EC,), _f32),
                     pltpu.VMEM((_TPB,), _f32),
                     pltpu.VMEM_SHARED((_NP,), _f32),
                     pltpu.VMEM_SHARED((_NP,), _f32),
                     pltpu.SemaphoreType.DMA],
  )
  c16, r16, degp, cntp = sc1(featp, colg, rowg, ewg, zrow)

  dinv, cnt, w3w1, c1 = pl.pallas_call(
      _prep_body,
      grid=(1,),
      in_specs=[_full((2, 80, 128)), _full((2, 80, 128)),
                _full((_HID, 288)), _full((288, 192)), _full((1, 288))],
      out_specs=[_full((80, 128)), _full((80, 128)),
                 _full((_HID, 192)), _full((1, 192))],
      out_shape=[jax.ShapeDtypeStruct((80, 128), _f32),
                 jax.ShapeDtypeStruct((80, 128), _f32),
                 jax.ShapeDtypeStruct((_HID, 192), _f32),
                 jax.ShapeDtypeStruct((1, 192), _f32)],
  )(degp.reshape(2, 80, 128), cntp.reshape(2, 80, 128), phi_w3, gcn_w1,
    phi_b3.reshape(1, 288))
  dinvc = dinv.reshape(_NP, 1)
  cntc = cnt.reshape(_NP, 1)
  wpa = w3w1[:, :128].astype(_bf16)
  wpb = _pad(w3w1[:, 128:].astype(_bf16), (_HID, 128))
  c1a = c1[:, :128]
  c1b = c1[:, 128:]

  ne = _E // _TE
  espec = pl.BlockSpec((_TE, 16), lambda i: (i, 0))
  mspec = pl.BlockSpec((_TE, 128), lambda i: (i, 0))
  ma, mb = pl.pallas_call(
      _phi_body,
      grid=(ne,),
      in_specs=[espec, espec, pl.BlockSpec((_TE, 1), lambda i: (i, 0)),
                _full((16, _HID)), _full((16, _HID)), _full((1, _HID)),
                _full((1, _HID)), _full((_HID, _HID)), _full((1, _HID)),
                _full((_HID, _HID)), _full((1, _HID)),
                _full((_HID, 128)), _full((_HID, 128))],
      out_specs=[mspec, mspec],
      out_shape=[jax.ShapeDtypeStruct((_E, 128), _f32),
                 jax.ShapeDtypeStruct((_E, 128), _f32)],
  )(c16, r16, edge_attr, w0c, w0r, w0e, b0, w1b, b1, w2b, b2, wpa, wpb)

  sc2 = pl.kernel(
      _sc_msg_body,
      out_type=[jax.ShapeDtypeStruct((_NP, 128), _f32),
                jax.ShapeDtypeStruct((_NP, 64), _f32)],
      mesh=_sc_mesh(),
      compiler_params=_SC_PARAMS,
      scratch_types=[pltpu.VMEM((4, _EC), jnp.int32),
                     pltpu.VMEM((_EC, 128), _f32),
                     pltpu.VMEM((_EC, 128), _f32),
                     pltpu.VMEM((_EC, 128), _f32),
                     pltpu.VMEM((_EC, 128), _f32),
                     pltpu.VMEM((_NHA // _NS // 3, 128), _f32),
                     pltpu.VMEM_SHARED((_NHA, 128), _f32),
                     pltpu.SemaphoreType.DMA],
  )
  h0a, h0b = sc2(ma, mb, colg, zblk)

  nn = _NP // _TE
  nspec = lambda w: pl.BlockSpec((_TE, w), lambda i: (i, 0))
  ta0, ta1, tb0, tb1 = pl.pallas_call(
      _g1_body,
      grid=(nn,),
      in_specs=[nspec(128), nspec(64), nspec(1), nspec(1),
                _full((1, 128)), _full((1, 64))],
      out_specs=[nspec(64), nspec(64), nspec(32), nspec(32)],
      out_shape=[jax.ShapeDtypeStruct((_NP, 64), _f32),
                 jax.ShapeDtypeStruct((_NP, 64), _f32),
                 jax.ShapeDtypeStruct((_NP, 32), _f32),
                 jax.ShapeDtypeStruct((_NP, 32), _f32)],
  )(h0a, h0b, dinvc, cntc, c1a, c1b)

  def spmm(dh, glo, ghi):
    return pl.kernel(
        functools.partial(_sc_spmm_body, dh),
        out_type=[jax.ShapeDtypeStruct((_NP, dh), _f32),
                  jax.ShapeDtypeStruct((_NP, dh), _f32)],
        mesh=_sc_mesh(),
        compiler_params=_SC_PARAMS,
        scratch_types=[pltpu.VMEM((4, _EC), jnp.int32),
                       pltpu.VMEM((4, _EC), jnp.int32),
                       pltpu.VMEM((_EC, dh), _f32),
                       pltpu.VMEM((_EC, dh), _f32),
                       pltpu.VMEM((_EC, dh), _f32),
                       pltpu.VMEM((_EC, dh), _f32),
                       pltpu.VMEM((4 * _EC,), _f32),
                       pltpu.VMEM((_TPB // 4, dh), _f32),
                       pltpu.VMEM_SHARED((_NP, dh), _f32),
                       pltpu.SemaphoreType.DMA],
    )(glo, ghi, colg, rowg, ewf, zblk)

  u0, u1 = spmm(64, ta0, ta1)
  u2, u3 = spmm(32, tb0, tb1)

  g2lo, g2hi = pl.pallas_call(
      _g2_body,
      grid=(nn,),
      in_specs=[nspec(64), nspec(64), nspec(32), nspec(32), nspec(1),
                _full((1, 64)), _full((1, 64)), _full((1, 32)),
                _full((1, 32)),
                _full((64, 64)), _full((64, 64)), _full((32, 64)),
                _full((32, 64)),
                _full((64, 64)), _full((64, 64)), _full((32, 64)),
                _full((32, 64))],
      out_specs=[nspec(64), nspec(64)],
      out_shape=[jax.ShapeDtypeStruct((_NP, 64), _f32),
                 jax.ShapeDtypeStruct((_NP, 64), _f32)],
  )(u0, u1, u2, u3, dinvc, *gb1, *w2qlo, *w2qhi)

  t2lo, t2hi = spmm(64, g2lo, g2hi)

  out = pl.pallas_call(
      _out_body,
      grid=(nn,),
      in_specs=[nspec(64), nspec(64), nspec(1),
                _full((64, 128)), _full((64, 128)), _full((1, 128))],
      out_specs=nspec(128),
      out_shape=jax.ShapeDtypeStruct((_NP, 128), _f32),
  )(t2lo, t2hi, dinvc, s0, s1, gb2)

  return out[:_N]


# speedup vs baseline: 1.2729x; 1.0097x over previous
import functools

import jax
import jax.numpy as jnp
from jax import lax
from jax.experimental import pallas as pl
from jax.experimental.pallas import tpu as pltpu
from jax.experimental.pallas import tpu_sc as plsc

_N = 10000
_NP = 10240
_NH = 5120
_NHA = 5136
_E = 160000
_ER = 1250
_EC = 128
_HID = 1024
_NS = 16
_TPB = _NP // _NS
_TE = 640

_f32 = jnp.float32
_bf16 = jnp.bfloat16
_SC_PARAMS = pltpu.CompilerParams(use_tc_tiling_on_sc=False,
                                  needs_layout_passes=False)


def _sc_mesh():
  return plsc.VectorSubcoreMesh(core_axis_name="c", subcore_axis_name="s")


def _sc_gather_body(feat_hbm, colg_hbm, rowg_hbm, ewg_hbm, zrow_hbm,
                    c16_hbm, r16_hbm, degp_hbm, cntp_hbm,
                    idxc_v, idxr_v, crows_v, rrows_v, ew_v, ones_v, z_v,
                    dacc_sh, cacc_sh, sem):
  cid = lax.axis_index("c")
  sid = lax.axis_index("s")
  def mkones(k, carry):
    ones_v[pl.ds(k * 16, 16)] = jnp.zeros((16,), _f32) + 1.0
    return carry
  lax.fori_loop(0, _EC // 16, mkones, 0)
  pltpu.sync_copy(zrow_hbm, z_v)
  pltpu.sync_copy(z_v, dacc_sh.at[pl.ds(sid * _TPB, _TPB)])
  pltpu.sync_copy(z_v, cacc_sh.at[pl.ds(sid * _TPB, _TPB)])
  plsc.subcore_barrier()

  half = _ER // 2
  npair = half // 2

  def dogather(idx_ref, j, dst, out_hbm, r):
    pltpu.async_copy(feat_hbm.at[idx_ref.at[j]], dst, sem).wait()
    pltpu.sync_copy(dst, out_hbm.at[pl.ds(r * _EC, _EC)])

  def body(g, carry):
    lp = sid + g * _NS

    @pl.when(lp < npair)
    def _():
      r = cid * half + lp * 2
      pltpu.sync_copy(colg_hbm.at[pl.ds(r, 2)], idxc_v)
      pltpu.sync_copy(rowg_hbm.at[pl.ds(r, 2)], idxr_v)
      pltpu.sync_copy(ewg_hbm.at[pl.ds(r, 2)], ew_v)
      for j in range(2):
        dogather(idxc_v, j, crows_v, c16_hbm, r + j)
        dogather(idxr_v, j, rrows_v, r16_hbm, r + j)
        pltpu.sync_copy(ew_v.at[j], dacc_sh.at[idxc_v.at[j]], add=True)
        pltpu.sync_copy(ones_v, cacc_sh.at[idxc_v.at[j]], add=True)
    return carry

  lax.fori_loop(0, (npair + _NS - 1) // _NS, body, 0)

  @pl.when(sid == 0)
  def _():
    r = cid * half + 2 * npair
    pltpu.sync_copy(colg_hbm.at[pl.ds(r, 1)], idxc_v.at[pl.ds(0, 1)])
    pltpu.sync_copy(rowg_hbm.at[pl.ds(r, 1)], idxr_v.at[pl.ds(0, 1)])
    pltpu.sync_copy(ewg_hbm.at[pl.ds(r, 1)], ew_v.at[pl.ds(0, 1)])
    dogather(idxc_v, 0, crows_v, c16_hbm, r)
    dogather(idxr_v, 0, rrows_v, r16_hbm, r)
    pltpu.sync_copy(ew_v.at[0], dacc_sh.at[idxc_v.at[0]], add=True)
    pltpu.sync_copy(ones_v, cacc_sh.at[idxc_v.at[0]], add=True)

  plsc.subcore_barrier()
  pltpu.sync_copy(dacc_sh.at[pl.ds(sid * _TPB, _TPB)],
                  degp_hbm.at[cid, pl.ds(sid * _TPB, _TPB)])
  pltpu.sync_copy(cacc_sh.at[pl.ds(sid * _TPB, _TPB)],
                  cntp_hbm.at[cid, pl.ds(sid * _TPB, _TPB)])


def _sc_msg_body(ma_hbm, mb_hbm, colg_hbm, zblk_hbm,
                 h0a_hbm, h0b_hbm,
                 idx_v, mr0_v, mr1_v, mr2_v, mr3_v, z_v, acc_sh, sem):
  cid = lax.axis_index("c")
  sid = lax.axis_index("s")
  rows_per_tile = _NHA // _NS
  pltpu.sync_copy(zblk_hbm.at[pl.ds(0, rows_per_tile // 3)], z_v)

  for p in range(2):
    base = p * _NH
    for tz in range(3):
      pltpu.sync_copy(
          z_v, acc_sh.at[pl.ds(sid * rows_per_tile + tz * 107, 107)])
    plsc.subcore_barrier()

    def remap(j, nk):
      for k in range(nk):
        v = idx_v[j, pl.ds(k * 16, 16)] - base
        ok = jnp.logical_and(v >= 0, v < _NH)
        idx_v[j, pl.ds(k * 16, 16)] = jnp.where(ok, v, _NH)

    ngrp = _ER // 4

    def body(g2, carry):
      g = sid + g2 * _NS

      @pl.when(g < ngrp)
      def _():
        pltpu.sync_copy(colg_hbm.at[pl.ds(g * 4, 4)], idx_v)
        for j in range(4):
          remap(j, _EC // 16)

        mrows = (mr0_v, mr1_v, mr2_v, mr3_v)
        for j in range(4):

          @pl.when(cid == 0)
          def _():
            pltpu.sync_copy(ma_hbm.at[pl.ds((g * 4 + j) * _EC, _EC)],
                            mrows[j])

          @pl.when(cid == 1)
          def _():
            pltpu.sync_copy(mb_hbm.at[pl.ds((g * 4 + j) * _EC, _EC)],
                            mrows[j])

        for j in range(4):
          pltpu.sync_copy(mrows[j], acc_sh.at[idx_v.at[j]], add=True)
      return carry

    lax.fori_loop(0, (ngrp + _NS - 1) // _NS, body, 0)

    @pl.when(sid < _ER - 4 * ngrp)
    def _():
      r = 4 * ngrp + sid
      pltpu.sync_copy(colg_hbm.at[r], idx_v.at[0])
      remap(0, _EC // 16)

      @pl.when(cid == 0)
      def _():
        pltpu.sync_copy(ma_hbm.at[pl.ds(r * _EC, _EC)], mr0_v)

      @pl.when(cid == 1)
      def _():
        pltpu.sync_copy(mb_hbm.at[pl.ds(r * _EC, _EC)], mr0_v)

      pltpu.sync_copy(mr0_v, acc_sh.at[idx_v.at[0]], add=True)

    plsc.subcore_barrier()
    out_rows = _NH // _NS
    src = pl.ds(sid * out_rows, out_rows)
    dst = pl.ds(base + sid * out_rows, out_rows)

    @pl.when(cid == 0)
    def _():
      pltpu.sync_copy(acc_sh.at[src], h0a_hbm.at[dst])

    @pl.when(cid == 1)
    def _():
      pltpu.sync_copy(acc_sh.at[src, pl.ds(0, 64)], h0b_hbm.at[dst])

    plsc.subcore_barrier()


def _sc_spmm_body(dh, glo_hbm, ghi_hbm, colg_hbm, rowg_hbm, ewf_hbm,
                  zblk_hbm, tlo_hbm, thi_hbm,
                  idxc_v, idxr_v, gr0_v, gr1_v, gr2_v, gr3_v, ew_v, z_v,
                  acc_sh, sem):
  cid = lax.axis_index("c")
  sid = lax.axis_index("s")
  pltpu.sync_copy(zblk_hbm.at[pl.ds(0, _TPB // 4), pl.ds(0, dh)], z_v)
  for tz in range(4):
    pltpu.sync_copy(z_v, acc_sh.at[pl.ds(sid * _TPB + tz * (_TPB // 4),
                                         _TPB // 4)])
  plsc.subcore_barrier()

  grows = (gr0_v, gr1_v, gr2_v, gr3_v)

  def escale(j, ebase):
    buf = grows[j]
    def one(e, c2):
      w = plsc.load_gather(ew_v, [jnp.zeros((16,), jnp.int32) + ebase + e])
      for q in range(dh // 16):
        buf[e, pl.ds(q * 16, 16)] = buf[e, pl.ds(q * 16, 16)] * w
      return c2
    lax.fori_loop(0, _EC, one, 0)

  def gather_rows(j):

    @pl.when(cid == 0)
    def _():
      pltpu.async_copy(glo_hbm.at[idxr_v.at[j]], grows[j], sem).wait()

    @pl.when(cid == 1)
    def _():
      pltpu.async_copy(ghi_hbm.at[idxr_v.at[j]], grows[j], sem).wait()

  ngrp = _ER // 4
  def body(k, carry):
    g = sid + k * _NS

    @pl.when(g < ngrp)
    def _():
      pltpu.sync_copy(colg_hbm.at[pl.ds(g * 4, 4)], idxc_v)
      pltpu.sync_copy(rowg_hbm.at[pl.ds(g * 4, 4)], idxr_v)
      pltpu.sync_copy(ewf_hbm.at[pl.ds(g * 4 * _EC, 4 * _EC)], ew_v)
      for j in range(4):
        gather_rows(j)
      for j in range(4):
        escale(j, j * _EC)
      for j in range(4):
        pltpu.sync_copy(grows[j], acc_sh.at[idxc_v.at[j]], add=True)
    return carry

  lax.fori_loop(0, (ngrp + _NS - 1) // _NS, body, 0)

  @pl.when(sid < _ER - 4 * ngrp)
  def _():
    r = 4 * ngrp + sid
    pltpu.sync_copy(colg_hbm.at[r], idxc_v.at[0])
    pltpu.sync_copy(rowg_hbm.at[r], idxr_v.at[0])
    pltpu.sync_copy(ewf_hbm.at[pl.ds(r * _EC, _EC)], ew_v.at[pl.ds(0, _EC)])
    gather_rows(0)
    escale(0, 0)
    pltpu.sync_copy(gr0_v, acc_sh.at[idxc_v.at[0]], add=True)

  plsc.subcore_barrier()
  sl = pl.ds(sid * _TPB, _TPB)

  @pl.when(cid == 0)
  def _():
    pltpu.sync_copy(acc_sh.at[sl], tlo_hbm.at[sl])

  @pl.when(cid == 1)
  def _():
    pltpu.sync_copy(acc_sh.at[sl], thi_hbm.at[sl])


def _prep_body(degp_ref, cntp_ref, w3_ref, gw1_ref, b3_ref,
               dinv_ref, cnt_ref, w3w1_ref, c1_ref):
  deg = degp_ref[0] + degp_ref[1]
  dinv_ref[...] = jnp.where(deg > 0, lax.rsqrt(jnp.maximum(deg, 1e-30)), 0.0)
  cnt_ref[...] = cntp_ref[0] + cntp_ref[1]
  w3w1_ref[...] = jnp.dot(w3_ref[...], gw1_ref[...],
                          preferred_element_type=_f32)
  c1_ref[...] = jnp.dot(b3_ref[...], gw1_ref[...],
                        preferred_element_type=_f32)


def _phi_body(c16_ref, r16_ref, ew_ref, w0c_ref, w0r_ref, w0e_ref, b0_ref,
              w1_ref, b1_ref, w2_ref, b2_ref, wpa_ref, wpb_ref,
              ma_ref, mb_ref):
  h = jnp.dot(c16_ref[...], w0c_ref[...], preferred_element_type=_f32)
  h = h + jnp.dot(r16_ref[...], w0r_ref[...], preferred_element_type=_f32)
  h = h + ew_ref[...] * w0e_ref[...]
  h = jax.nn.relu(h + b0_ref[...]).astype(_bf16)
  h = jax.nn.relu(jnp.dot(h, w1_ref[...], preferred_element_type=_f32)
                  + b1_ref[...]).astype(_bf16)
  h = jax.nn.relu(jnp.dot(h, w2_ref[...], preferred_element_type=_f32)
                  + b2_ref[...]).astype(_bf16)
  ma_ref[...] = jnp.dot(h, wpa_ref[...], preferred_element_type=_f32)
  mb_ref[...] = jnp.dot(h, wpb_ref[...], preferred_element_type=_f32)


def _g1_body(h0a_ref, h0b_ref, dinv_ref, cnt_ref, c1a_ref, c1b_ref,
             ta0_ref, ta1_ref, tb0_ref, tb1_ref):
  d = dinv_ref[...]
  cnt = cnt_ref[...]
  va = d * (h0a_ref[...] + cnt * c1a_ref[...])
  vb = d * (h0b_ref[...] + cnt * c1b_ref[...])
  ta0_ref[...] = va[:, :64]
  ta1_ref[...] = va[:, 64:]
  tb0_ref[...] = vb[:, :32]
  tb1_ref[...] = vb[:, 32:]


def _g2_body(u0_ref, u1_ref, u2_ref, u3_ref, dinv_ref,
             b1a_ref, b1b_ref, b1c_ref, b1d_ref,
             wa_lo, wb_lo, wc_lo, wd_lo, wa_hi, wb_hi, wc_hi, wd_hi,
             glo_ref, ghi_ref):
  d = dinv_ref[...]
  h1 = [jax.nn.relu(d * t[...] + b[...])
        for t, b in ((u0_ref, b1a_ref), (u1_ref, b1b_ref),
                     (u2_ref, b1c_ref), (u3_ref, b1d_ref))]
  wlo = (wa_lo, wb_lo, wc_lo, wd_lo)
  whi = (wa_hi, wb_hi, wc_hi, wd_hi)
  glo = jnp.dot(h1[0], wlo[0][...], preferred_element_type=_f32)
  ghi = jnp.dot(h1[0], whi[0][...], preferred_element_type=_f32)
  for q in range(1, 4):
    glo = glo + jnp.dot(h1[q], wlo[q][...], preferred_element_type=_f32)
    ghi = ghi + jnp.dot(h1[q], whi[q][...], preferred_element_type=_f32)
  glo_ref[...] = d * glo
  ghi_ref[...] = d * ghi


def _out_body(t2lo_ref, t2hi_ref, dinv_ref, s0_ref, s1_ref, b2_ref, out_ref):
  d = dinv_ref[...]
  t2 = jnp.dot(t2lo_ref[...], s0_ref[...], preferred_element_type=_f32) \
      + jnp.dot(t2hi_ref[...], s1_ref[...], preferred_element_type=_f32)
  out_ref[...] = jax.nn.relu(d * t2 + b2_ref[...])


def _full(shape):
  return pl.BlockSpec(shape, lambda *_: tuple(0 for _ in shape))


def _pad(a, shape):
  out = jnp.zeros(shape, a.dtype)
  return out.at[tuple(slice(0, s) for s in a.shape)].set(a)


def kernel(x, edge_attr, edge_index, emb_table,
           phi_w0, phi_b0, phi_w1, phi_b1, phi_w2, phi_b2, phi_w3, phi_b3,
           gcn_w1, gcn_b1, gcn_w2, gcn_b2):
  nf = jnp.stack([x[:, 0], x[:, 2], x[:, 3], x[:, 4]], axis=1)
  t = x[:, 1].astype(jnp.int32)
  emb = jnp.where((t == 0)[:, None], emb_table[0][None, :],
                  emb_table[1][None, :])
  featp = _pad(jnp.concatenate([nf, emb], axis=1), (_NP, 16))

  rowg = edge_index[0].reshape(_ER, _EC)
  colg = edge_index[1].reshape(_ER, _EC)
  ewf = edge_attr[:, 0]
  ewg = ewf.reshape(_ER, _EC)
  zrow = jnp.zeros((_TPB,), _f32)
  zblk = jnp.zeros((_TPB, 128), _f32)

  w0c = _pad(phi_w0[:6], (16, _HID))
  w0r = _pad(phi_w0[6:12], (16, _HID))
  w0e = phi_w0[12].reshape(1, _HID)
  b0 = phi_b0.reshape(1, _HID)
  b1 = phi_b1.reshape(1, _HID)
  b2 = phi_b2.reshape(1, _HID)
  w1b = phi_w1.astype(_bf16)
  w2b = phi_w2.astype(_bf16)
  gb1 = [gcn_b1[0:64].reshape(1, 64), gcn_b1[64:128].reshape(1, 64),
         gcn_b1[128:160].reshape(1, 32), gcn_b1[160:192].reshape(1, 32)]
  w2rows = [gcn_w2[0:64], gcn_w2[64:128], gcn_w2[128:160], gcn_w2[160:192]]
  w2qlo = [w[:, :64] for w in w2rows]
  w2qhi = [w[:, 64:] for w in w2rows]
  gb2 = gcn_b2.reshape(1, 128)
  eye64 = jnp.eye(64, dtype=_f32)
  s0 = jnp.concatenate([eye64, jnp.zeros((64, 64), _f32)], axis=1)
  s1 = jnp.concatenate([jnp.zeros((64, 64), _f32), eye64], axis=1)

  sc1 = pl.kernel(
      _sc_gather_body,
      out_type=[jax.ShapeDtypeStruct((_E, 16), _f32),
                jax.ShapeDtypeStruct((_E, 16), _f32),
                jax.ShapeDtypeStruct((2, _NP), _f32),
                jax.ShapeDtypeStruct((2, _NP), _f32)],
      mesh=_sc_mesh(),
      compiler_params=_SC_PARAMS,
      scratch_types=[pltpu.VMEM((2, _EC), jnp.int32),
                     pltpu.VMEM((2, _EC), jnp.int32),
                     pltpu.VMEM((_EC, 16), _f32),
                     pltpu.VMEM((_EC, 16), _f32),
                     pltpu.VMEM((2, _EC), _f32),
                     pltpu.VMEM((_EC,), _f32),
                     pltpu.VMEM((_TPB,), _f32),
                     pltpu.VMEM_SHARED((_NP,), _f32),
                     pltpu.VMEM_SHARED((_NP,), _f32),
                     pltpu.SemaphoreType.DMA],
  )
  c16, r16, degp, cntp = sc1(featp, colg, rowg, ewg, zrow)

  dinv, cnt, w3w1, c1 = pl.pallas_call(
      _prep_body,
      grid=(1,),
      in_specs=[_full((2, 80, 128)), _full((2, 80, 128)),
                _full((_HID, 288)), _full((288, 192)), _full((1, 288))],
      out_specs=[_full((80, 128)), _full((80, 128)),
                 _full((_HID, 192)), _full((1, 192))],
      out_shape=[jax.ShapeDtypeStruct((80, 128), _f32),
                 jax.ShapeDtypeStruct((80, 128), _f32),
                 jax.ShapeDtypeStruct((_HID, 192), _f32),
                 jax.ShapeDtypeStruct((1, 192), _f32)],
  )(degp.reshape(2, 80, 128), cntp.reshape(2, 80, 128), phi_w3, gcn_w1,
    phi_b3.reshape(1, 288))
  dinvc = dinv.reshape(_NP, 1)
  cntc = cnt.reshape(_NP, 1)
  wpa = w3w1[:, :128].astype(_bf16)
  wpb = _pad(w3w1[:, 128:].astype(_bf16), (_HID, 128))
  c1a = c1[:, :128]
  c1b = c1[:, 128:]

  ne = _E // _TE
  espec = pl.BlockSpec((_TE, 16), lambda i: (i, 0))
  mspec = pl.BlockSpec((_TE, 128), lambda i: (i, 0))
  ma, mb = pl.pallas_call(
      _phi_body,
      grid=(ne,),
      in_specs=[espec, espec, pl.BlockSpec((_TE, 1), lambda i: (i, 0)),
                _full((16, _HID)), _full((16, _HID)), _full((1, _HID)),
                _full((1, _HID)), _full((_HID, _HID)), _full((1, _HID)),
                _full((_HID, _HID)), _full((1, _HID)),
                _full((_HID, 128)), _full((_HID, 128))],
      out_specs=[mspec, mspec],
      out_shape=[jax.ShapeDtypeStruct((_E, 128), _f32),
                 jax.ShapeDtypeStruct((_E, 128), _f32)],
  )(c16, r16, edge_attr, w0c, w0r, w0e, b0, w1b, b1, w2b, b2, wpa, wpb)

  sc2 = pl.kernel(
      _sc_msg_body,
      out_type=[jax.ShapeDtypeStruct((_NP, 128), _f32),
                jax.ShapeDtypeStruct((_NP, 64), _f32)],
      mesh=_sc_mesh(),
      compiler_params=_SC_PARAMS,
      scratch_types=[pltpu.VMEM((4, _EC), jnp.int32),
                     pltpu.VMEM((_EC, 128), _f32),
                     pltpu.VMEM((_EC, 128), _f32),
                     pltpu.VMEM((_EC, 128), _f32),
                     pltpu.VMEM((_EC, 128), _f32),
                     pltpu.VMEM((_NHA // _NS // 3, 128), _f32),
                     pltpu.VMEM_SHARED((_NHA, 128), _f32),
                     pltpu.SemaphoreType.DMA],
  )
  h0a, h0b = sc2(ma, mb, colg, zblk)

  nn = _NP // _TE
  nspec = lambda w: pl.BlockSpec((_TE, w), lambda i: (i, 0))
  ta0, ta1, tb0, tb1 = pl.pallas_call(
      _g1_body,
      grid=(nn,),
      in_specs=[nspec(128), nspec(64), nspec(1), nspec(1),
                _full((1, 128)), _full((1, 64))],
      out_specs=[nspec(64), nspec(64), nspec(32), nspec(32)],
      out_shape=[jax.ShapeDtypeStruct((_NP, 64), _f32),
                 jax.ShapeDtypeStruct((_NP, 64), _f32),
                 jax.ShapeDtypeStruct((_NP, 32), _f32),
                 jax.ShapeDtypeStruct((_NP, 32), _f32)],
  )(h0a, h0b, dinvc, cntc, c1a, c1b)

  def spmm(dh, glo, ghi):
    return pl.kernel(
        functools.partial(_sc_spmm_body, dh),
        out_type=[jax.ShapeDtypeStruct((_NP, dh), _f32),
                  jax.ShapeDtypeStruct((_NP, dh), _f32)],
        mesh=_sc_mesh(),
        compiler_params=_SC_PARAMS,
        scratch_types=[pltpu.VMEM((4, _EC), jnp.int32),
                       pltpu.VMEM((4, _EC), jnp.int32),
                       pltpu.VMEM((_EC, dh), _f32),
                       pltpu.VMEM((_EC, dh), _f32),
                       pltpu.VMEM((_EC, dh), _f32),
                       pltpu.VMEM((_EC, dh), _f32),
                       pltpu.VMEM((4 * _EC,), _f32),
                       pltpu.VMEM((_TPB // 4, dh), _f32),
                       pltpu.VMEM_SHARED((_NP, dh), _f32),
                       pltpu.SemaphoreType.DMA],
    )(glo, ghi, colg, rowg, ewf, zblk)

  u0, u1 = spmm(64, ta0, ta1)
  u2, u3 = spmm(32, tb0, tb1)

  g2lo, g2hi = pl.pallas_call(
      _g2_body,
      grid=(nn,),
      in_specs=[nspec(64), nspec(64), nspec(32), nspec(32), nspec(1),
                _full((1, 64)), _full((1, 64)), _full((1, 32)),
                _full((1, 32)),
                _full((64, 64)), _full((64, 64)), _full((32, 64)),
                _full((32, 64)),
                _full((64, 64)), _full((64, 64)), _full((32, 64)),
                _full((32, 64))],
      out_specs=[nspec(64), nspec(64)],
      out_shape=[jax.ShapeDtypeStruct((_NP, 64), _f32),
                 jax.ShapeDtypeStruct((_NP, 64), _f32)],
  )(u0, u1, u2, u3, dinvc, *gb1, *w2qlo, *w2qhi)

  t2lo, t2hi = spmm(64, g2lo, g2hi)

  out = pl.pallas_call(
      _out_body,
      grid=(nn,),
      in_specs=[nspec(64), nspec(64), nspec(1),
                _full((64, 128)), _full((64, 128)), _full((1, 128))],
      out_specs=nspec(128),
      out_shape=jax.ShapeDtypeStruct((_NP, 128), _f32),
  )(t2lo, t2hi, dinvc, s0, s1, gb2)

  return out[:_N]
